# baseline (speedup 1.0000x reference)
import functools

import jax
import jax.numpy as jnp
from jax import lax
from jax.experimental import pallas as pl
from jax.experimental.pallas import tpu as pltpu
from jax.experimental.pallas import tpu_sc as plsc

N_NODES = 10000
N_EDGES = 320000
D = 128

NC = 2
NS = 16
NW = NC * NS
N_PAD = 10112
ROWS_PER_TILE = N_PAD // NS
PAD_SPREAD = N_PAD - N_NODES

EPT = N_EDGES // NW
CHUNK = 112
NCHUNK = 90
EPT_PAD = NCHUNK * CHUNK
PHASE = 15
NPHASE = NCHUNK // PHASE
NBUF = 3

_sc_mesh = plsc.VectorSubcoreMesh(core_axis_name="c", subcore_axis_name="s")


def _deg_body(dst_hbm, zeros_hbm, out_hbm, idx_v, ones_v, acc_sh):
    c = lax.axis_index("c")
    s = lax.axis_index("s")
    for j in range(CHUNK // 16):
        ones_v[pl.ds(j * 16, 16)] = jnp.ones((16,), jnp.float32)

    @pl.when(s == 0)
    def _():
        pltpu.sync_copy(zeros_hbm, acc_sh)

    wid = c * NS + s
    pltpu.sync_copy(dst_hbm.at[wid], idx_v)
    plsc.subcore_barrier()

    def step(i, carry):
        pltpu.sync_copy(ones_v, acc_sh.at[idx_v.at[i]], add=True)
        return carry

    lax.fori_loop(0, NCHUNK, step, 0)
    plsc.subcore_barrier()

    @pl.when(s == 0)
    def _():
        pltpu.sync_copy(acc_sh, out_hbm.at[c])


_deg_kernel = functools.partial(
    pl.kernel,
    out_type=jax.ShapeDtypeStruct((NC, N_PAD), jnp.float32),
    mesh=_sc_mesh,
    scratch_types=[
        pltpu.VMEM((NCHUNK, CHUNK), jnp.int32),
        pltpu.VMEM((CHUNK,), jnp.float32),
        pltpu.VMEM_SHARED((N_PAD,), jnp.float32),
    ],
)(_deg_body)


def _agg_body(u_hbm, src_hbm, dst_hbm, zeros_hbm, out_hbm,
              src_v, dst_v, rows0, rows1, rows2, acc_sh,
              g0, g1, g2, t0, t1, t2):
    rows = [rows0, rows1, rows2]
    gsem = [g0, g1, g2]
    ssem = [t0, t1, t2]
    c = lax.axis_index("c")
    s = lax.axis_index("s")
    r0 = s * ROWS_PER_TILE
    pltpu.sync_copy(zeros_hbm.at[pl.ds(r0, ROWS_PER_TILE)],
                    acc_sh.at[pl.ds(r0, ROWS_PER_TILE)])
    wid = c * NS + s
    plsc.subcore_barrier()

    def phase(p, carry):
        pltpu.sync_copy(src_hbm.at[wid, p], src_v)
        pltpu.sync_copy(dst_hbm.at[wid, p], dst_v)

        def triple(k, carry2):
            i = k * NBUF
            gathers = [
                pltpu.async_copy(u_hbm.at[src_v.at[i + b]], rows[b], gsem[b])
                for b in range(NBUF)
            ]
            scatters = []
            for b in range(NBUF):
                gathers[b].wait()
                scatters.append(pltpu.async_copy(
                    rows[b], acc_sh.at[dst_v.at[i + b]], ssem[b], add=True))
            for b in range(NBUF):
                scatters[b].wait()
            return carry2

        lax.fori_loop(0, PHASE // NBUF, triple, 0)
        return carry

    lax.fori_loop(0, NPHASE, phase, 0)
    plsc.subcore_barrier()
    pltpu.sync_copy(acc_sh.at[pl.ds(r0, ROWS_PER_TILE)],
                    out_hbm.at[c, pl.ds(r0, ROWS_PER_TILE)])


_agg_kernel = functools.partial(
    pl.kernel,
    out_type=jax.ShapeDtypeStruct((NC, N_PAD, D), jnp.float32),
    mesh=_sc_mesh,
    scratch_types=[
        pltpu.VMEM((PHASE, CHUNK), jnp.int32),
        pltpu.VMEM((PHASE, CHUNK), jnp.int32),
        pltpu.VMEM((CHUNK, D), jnp.float32),
        pltpu.VMEM((CHUNK, D), jnp.float32),
        pltpu.VMEM((CHUNK, D), jnp.float32),
        pltpu.VMEM_SHARED((N_PAD, D), jnp.float32),
        pltpu.SemaphoreType.DMA,
        pltpu.SemaphoreType.DMA,
        pltpu.SemaphoreType.DMA,
        pltpu.SemaphoreType.DMA,
        pltpu.SemaphoreType.DMA,
        pltpu.SemaphoreType.DMA,
    ],
)(_agg_body)


ROW_BLK = 1264
GRID = N_PAD // ROW_BLK


def _dinv_of(degp_blk):
    deg = degp_blk[:, 0] + degp_blk[:, 1] + 1.0
    return lax.rsqrt(deg)


def _tc0_body(x_ref, w_ref, h_ref):
    h_ref[...] = jnp.dot(x_ref[...], w_ref[...],
                         preferred_element_type=jnp.float32)


def _tc1_body(h_ref, degp_ref, u_ref):
    dinv = _dinv_of(degp_ref)
    u_ref[...] = h_ref[...] * dinv[:, None]


def _tc2_body(aggp_ref, u1_ref, degp_ref, w_ref, b_ref, u2_ref):
    dinv = _dinv_of(degp_ref)
    enc = (aggp_ref[0] + aggp_ref[1] + u1_ref[...]) * dinv[:, None] + b_ref[...]
    h2 = jnp.dot(enc, w_ref[...], preferred_element_type=jnp.float32)
    u2_ref[...] = h2 * dinv[:, None]


def _tc3_body(aggp_ref, u2_ref, degp_ref, b_ref, out_ref):
    dinv = _dinv_of(degp_ref)
    out_ref[...] = ((aggp_ref[0] + aggp_ref[1] + u2_ref[...]) * dinv[:, None]
                    + b_ref[...])


_rowblk = pl.BlockSpec((ROW_BLK, D), lambda i: (i, 0))
_wblk = pl.BlockSpec((D, D), lambda i: (0, 0))
_bblk = pl.BlockSpec((1, D), lambda i: (0, 0))
_degblk = pl.BlockSpec((ROW_BLK, NC), lambda i: (i, 0))
_aggblk = pl.BlockSpec((NC, ROW_BLK, D), lambda i: (0, i, 0))
_out_rows = jax.ShapeDtypeStruct((N_PAD, D), jnp.float32)

_tc0 = pl.pallas_call(
    _tc0_body,
    grid=(GRID,),
    in_specs=[_rowblk, _wblk],
    out_specs=_rowblk,
    out_shape=_out_rows,
)

_tc1 = pl.pallas_call(
    _tc1_body,
    grid=(GRID,),
    in_specs=[_rowblk, _degblk],
    out_specs=_rowblk,
    out_shape=_out_rows,
)

_tc2 = pl.pallas_call(
    _tc2_body,
    grid=(GRID,),
    in_specs=[_aggblk, _rowblk, _degblk, _wblk, _bblk],
    out_specs=_rowblk,
    out_shape=_out_rows,
)

_tc3 = pl.pallas_call(
    _tc3_body,
    grid=(GRID,),
    in_specs=[_aggblk, _rowblk, _degblk, _bblk],
    out_specs=_rowblk,
    out_shape=_out_rows,
)


def _pad_edges(e):
    e2 = e.reshape(NW, EPT)
    k = jnp.arange(NW * (EPT_PAD - EPT), dtype=jnp.int32)
    pad = (N_NODES + k % PAD_SPREAD).reshape(NW, EPT_PAD - EPT)
    return jnp.concatenate([e2, pad], axis=1)


def kernel(x, edge_index, W_enc, b_enc, W_dec, b_dec):
    src_flat = _pad_edges(edge_index[0].astype(jnp.int32))
    dst_flat = _pad_edges(edge_index[1].astype(jnp.int32))
    src_p = src_flat.reshape(NW, NPHASE, PHASE, CHUNK)
    dst_p = dst_flat.reshape(NW, NPHASE, PHASE, CHUNK)
    dst_deg = dst_flat.reshape(NW, NCHUNK, CHUNK)
    b_enc2 = b_enc.reshape(1, D)
    b_dec2 = b_dec.reshape(1, D)
    zeros1 = jnp.zeros((N_PAD,), jnp.float32)
    zeros2 = jnp.zeros((N_PAD, D), jnp.float32)
    x_pad = jnp.concatenate(
        [x, jnp.zeros((N_PAD - N_NODES, D), jnp.float32)], axis=0)

    h1 = _tc0(x_pad, W_enc)
    degp = _deg_kernel(dst_deg, zeros1).T
    u1 = _tc1(h1, degp)
    agg1 = _agg_kernel(u1, src_p, dst_p, zeros2)
    u2 = _tc2(agg1, u1, degp, W_dec, b_enc2)
    agg2 = _agg_kernel(u2, src_p, dst_p, zeros2)
    return _tc3(agg2, u2, degp, b_dec2)[:N_NODES]


# speedup vs baseline: 1.0204x; 1.0204x over previous
import functools

import jax
import jax.numpy as jnp
from jax import lax
from jax.experimental import pallas as pl
from jax.experimental.pallas import tpu as pltpu
from jax.experimental.pallas import tpu_sc as plsc

N_NODES = 10000
N_EDGES = 320000
D = 128

NC = 2
NS = 16
NW = NC * NS
N_PAD = 10112
ROWS_PER_TILE = N_PAD // NS
PAD_SPREAD = N_PAD - N_NODES

EPT = N_EDGES // NW
CHUNK = 112
NCHUNK = 90
EPT_PAD = NCHUNK * CHUNK
PHASE = 15
NPHASE = NCHUNK // PHASE
NBUF = 3

_sc_mesh = plsc.VectorSubcoreMesh(core_axis_name="c", subcore_axis_name="s")


def _deg_body(dst_hbm, zeros_hbm, out_hbm, idx_v, ones_v, acc_sh,
              d0, d1, d2):
    dsem = [d0, d1, d2]
    c = lax.axis_index("c")
    s = lax.axis_index("s")
    for j in range(CHUNK // 16):
        ones_v[pl.ds(j * 16, 16)] = jnp.ones((16,), jnp.float32)

    @pl.when(s == 0)
    def _():
        pltpu.sync_copy(zeros_hbm, acc_sh)

    wid = c * NS + s
    pltpu.sync_copy(dst_hbm.at[wid], idx_v)
    plsc.subcore_barrier()

    def step(k, carry):
        i = k * NBUF
        scatters = [
            pltpu.async_copy(ones_v, acc_sh.at[idx_v.at[i + b]], dsem[b],
                             add=True)
            for b in range(NBUF)
        ]
        for b in range(NBUF):
            scatters[b].wait()
        return carry

    lax.fori_loop(0, NCHUNK // NBUF, step, 0)
    plsc.subcore_barrier()

    @pl.when(s == 0)
    def _():
        pltpu.sync_copy(acc_sh, out_hbm.at[c])


_deg_kernel = functools.partial(
    pl.kernel,
    out_type=jax.ShapeDtypeStruct((NC, N_PAD), jnp.float32),
    mesh=_sc_mesh,
    scratch_types=[
        pltpu.VMEM((NCHUNK, CHUNK), jnp.int32),
        pltpu.VMEM((CHUNK,), jnp.float32),
        pltpu.VMEM_SHARED((N_PAD,), jnp.float32),
        pltpu.SemaphoreType.DMA,
        pltpu.SemaphoreType.DMA,
        pltpu.SemaphoreType.DMA,
    ],
)(_deg_body)


def _agg_body(u_hbm, src_hbm, dst_hbm, zeros_hbm, out_hbm,
              src_v, dst_v, rows0, rows1, rows2, acc_sh,
              g0, g1, g2, t0, t1, t2):
    rows = [rows0, rows1, rows2]
    gsem = [g0, g1, g2]
    ssem = [t0, t1, t2]
    c = lax.axis_index("c")
    s = lax.axis_index("s")
    r0 = s * ROWS_PER_TILE
    pltpu.sync_copy(zeros_hbm.at[pl.ds(r0, ROWS_PER_TILE)],
                    acc_sh.at[pl.ds(r0, ROWS_PER_TILE)])
    wid = c * NS + s
    plsc.subcore_barrier()

    def phase(p, carry):
        pltpu.sync_copy(src_hbm.at[wid, p], src_v)
        pltpu.sync_copy(dst_hbm.at[wid, p], dst_v)

        def triple(k, carry2):
            i = k * NBUF
            gathers = [
                pltpu.async_copy(u_hbm.at[src_v.at[i + b]], rows[b], gsem[b])
                for b in range(NBUF)
            ]
            scatters = []
            for b in range(NBUF):
                gathers[b].wait()
                scatters.append(pltpu.async_copy(
                    rows[b], acc_sh.at[dst_v.at[i + b]], ssem[b], add=True))
            for b in range(NBUF):
                scatters[b].wait()
            return carry2

        lax.fori_loop(0, PHASE // NBUF, triple, 0)
        return carry

    lax.fori_loop(0, NPHASE, phase, 0)
    plsc.subcore_barrier()
    pltpu.sync_copy(acc_sh.at[pl.ds(r0, ROWS_PER_TILE)],
                    out_hbm.at[c, pl.ds(r0, ROWS_PER_TILE)])


_agg_kernel = functools.partial(
    pl.kernel,
    out_type=jax.ShapeDtypeStruct((NC, N_PAD, D), jnp.float32),
    mesh=_sc_mesh,
    scratch_types=[
        pltpu.VMEM((PHASE, CHUNK), jnp.int32),
        pltpu.VMEM((PHASE, CHUNK), jnp.int32),
        pltpu.VMEM((CHUNK, D), jnp.float32),
        pltpu.VMEM((CHUNK, D), jnp.float32),
        pltpu.VMEM((CHUNK, D), jnp.float32),
        pltpu.VMEM_SHARED((N_PAD, D), jnp.float32),
        pltpu.SemaphoreType.DMA,
        pltpu.SemaphoreType.DMA,
        pltpu.SemaphoreType.DMA,
        pltpu.SemaphoreType.DMA,
        pltpu.SemaphoreType.DMA,
        pltpu.SemaphoreType.DMA,
    ],
)(_agg_body)


ROW_BLK = 1264
GRID = N_PAD // ROW_BLK


def _dinv_of(degp_blk):
    deg = degp_blk[:, 0] + degp_blk[:, 1] + 1.0
    return lax.rsqrt(deg)


def _tc1_body(x_ref, w_ref, degp_ref, u_ref):
    dinv = _dinv_of(degp_ref)
    h = jnp.dot(x_ref[...], w_ref[...], preferred_element_type=jnp.float32)
    u_ref[...] = h * dinv[:, None]


def _tc2_body(aggp_ref, u1_ref, degp_ref, w_ref, b_ref, u2_ref):
    dinv = _dinv_of(degp_ref)
    enc = (aggp_ref[0] + aggp_ref[1] + u1_ref[...]) * dinv[:, None] + b_ref[...]
    h2 = jnp.dot(enc, w_ref[...], preferred_element_type=jnp.float32)
    u2_ref[...] = h2 * dinv[:, None]


def _tc3_body(aggp_ref, u2_ref, degp_ref, b_ref, out_ref):
    dinv = _dinv_of(degp_ref)
    out_ref[...] = ((aggp_ref[0] + aggp_ref[1] + u2_ref[...]) * dinv[:, None]
                    + b_ref[...])


_rowblk = pl.BlockSpec((ROW_BLK, D), lambda i: (i, 0))
_wblk = pl.BlockSpec((D, D), lambda i: (0, 0))
_bblk = pl.BlockSpec((1, D), lambda i: (0, 0))
_degblk = pl.BlockSpec((ROW_BLK, NC), lambda i: (i, 0))
_aggblk = pl.BlockSpec((NC, ROW_BLK, D), lambda i: (0, i, 0))
_out_rows = jax.ShapeDtypeStruct((N_PAD, D), jnp.float32)

_tc1 = pl.pallas_call(
    _tc1_body,
    grid=(GRID,),
    in_specs=[_rowblk, _wblk, _degblk],
    out_specs=_rowblk,
    out_shape=_out_rows,
)

_tc2 = pl.pallas_call(
    _tc2_body,
    grid=(GRID,),
    in_specs=[_aggblk, _rowblk, _degblk, _wblk, _bblk],
    out_specs=_rowblk,
    out_shape=_out_rows,
)

_tc3 = pl.pallas_call(
    _tc3_body,
    grid=(GRID,),
    in_specs=[_aggblk, _rowblk, _degblk, _bblk],
    out_specs=_rowblk,
    out_shape=_out_rows,
)


def _pad_edges(e):
    e2 = e.reshape(NW, EPT)
    k = jnp.arange(NW * (EPT_PAD - EPT), dtype=jnp.int32)
    pad = (N_NODES + k % PAD_SPREAD).reshape(NW, EPT_PAD - EPT)
    return jnp.concatenate([e2, pad], axis=1)


def kernel(x, edge_index, W_enc, b_enc, W_dec, b_dec):
    src_flat = _pad_edges(edge_index[0].astype(jnp.int32))
    dst_flat = _pad_edges(edge_index[1].astype(jnp.int32))
    src_p = src_flat.reshape(NW, NPHASE, PHASE, CHUNK)
    dst_p = dst_flat.reshape(NW, NPHASE, PHASE, CHUNK)
    dst_deg = dst_flat.reshape(NW, NCHUNK, CHUNK)
    b_enc2 = b_enc.reshape(1, D)
    b_dec2 = b_dec.reshape(1, D)
    zeros1 = jnp.zeros((N_PAD,), jnp.float32)
    zeros2 = jnp.zeros((N_PAD, D), jnp.float32)
    x_pad = jnp.concatenate(
        [x, jnp.zeros((N_PAD - N_NODES, D), jnp.float32)], axis=0)

    degp = _deg_kernel(dst_deg, zeros1).T
    u1 = _tc1(x_pad, W_enc, degp)
    agg1 = _agg_kernel(u1, src_p, dst_p, zeros2)
    u2 = _tc2(agg1, u1, degp, W_dec, b_enc2)
    agg2 = _agg_kernel(u2, src_p, dst_p, zeros2)
    return _tc3(agg2, u2, degp, b_dec2)[:N_NODES]
